# baseline (speedup 1.0000x reference)
import functools

import jax
import jax.numpy as jnp
from jax import lax
from jax.experimental import pallas as pl
from jax.experimental.pallas import tpu as pltpu
from jax.experimental.pallas import tpu_sc as plsc

DIMS = (64, 64, 64)
FEAT = 64
BATCH = 16384
NROWS = DIMS[0] * DIMS[1] * DIMS[2]

NUM_CORES = 2
NUM_SUBCORES = 16
LANES = 16
NUM_WORKERS = NUM_CORES * NUM_SUBCORES
BPW = BATCH // NUM_WORKERS
CHUNK = 128
NCHUNK = BPW // CHUNK

_mesh = plsc.VectorSubcoreMesh(core_axis_name="c", subcore_axis_name="s")


@functools.partial(
    pl.kernel,
    mesh=_mesh,
    compiler_params=pltpu.CompilerParams(use_tc_tiling_on_sc=False),
    out_type=jax.ShapeDtypeStruct((BATCH, FEAT), jnp.float32),
    scratch_types=[
        pltpu.VMEM((BPW * 3,), jnp.int32),
        pltpu.VMEM((NCHUNK, CHUNK), jnp.int32),
        pltpu.VMEM((BPW, FEAT), jnp.float32),
        pltpu.SemaphoreType.DMA,
    ],
)
def _lookup(idx_hbm, tab_hbm, out_hbm, raw_v, flat_v, rows_v, sem):
    wid = lax.axis_index("s") * NUM_CORES + lax.axis_index("c")
    base = wid * BPW

    pltpu.sync_copy(idx_hbm.at[pl.ds(base, BPW)], raw_v.at[pl.ds(0, BPW)])
    pltpu.sync_copy(idx_hbm.at[pl.ds(BATCH + base, BPW)],
                    raw_v.at[pl.ds(BPW, BPW)])
    pltpu.sync_copy(idx_hbm.at[pl.ds(2 * BATCH + base, BPW)],
                    raw_v.at[pl.ds(2 * BPW, BPW)])

    for g in range(BPW // LANES):
        o16 = g * LANES
        i0 = raw_v[pl.ds(o16, LANES)]
        i1 = raw_v[pl.ds(BPW + o16, LANES)]
        i2 = raw_v[pl.ds(2 * BPW + o16, LANES)]
        flat = i0 * (DIMS[1] * DIMS[2]) + i1 * DIMS[2] + i2
        c, o = divmod(o16, CHUNK)
        flat_v[c, pl.ds(o, LANES)] = flat

    copies = [
        pltpu.async_copy(
            tab_hbm.at[flat_v.at[c]],
            rows_v.at[pl.ds(c * CHUNK, CHUNK)],
            sem,
        )
        for c in range(NCHUNK)
    ]
    for cp in copies:
        cp.wait()

    pltpu.sync_copy(rows_v, out_hbm.at[pl.ds(base, BPW)])


def kernel(indices, table):
    idx_cols = indices.astype(jnp.int32).T.reshape(-1)
    tab2d = table.reshape(NROWS, FEAT)
    return _lookup(idx_cols, tab2d)
